# baseline (speedup 1.0000x reference)
import functools

import jax
import jax.numpy as jnp
from jax import lax
from jax.experimental import pallas as pl
from jax.experimental.pallas import tpu as pltpu
from jax.experimental.pallas import tpu_sc as plsc

_E = 8
_K = 2
_BLK = 256
_NB = 24
_NW = 32


def _router(x, wrt, bias):
    N, _ = x.shape

    def body(x_ref, w_ref, b_ref, idx_ref, wt_ref):
        y = lax.dot_general(x_ref[...], w_ref[...], (((1,), (0,)), ((), ())),
                            preferred_element_type=jnp.float32) + b_ref[...]
        iota = lax.broadcasted_iota(jnp.int32, y.shape, 1)
        m0 = jnp.max(y, axis=1, keepdims=True)
        i0 = jnp.min(jnp.where(y == m0, iota, _E), axis=1)
        ymask = jnp.where(iota == i0[:, None], -1e30, y)
        m1 = jnp.max(ymask, axis=1, keepdims=True)
        i1 = jnp.min(jnp.where(ymask == m1, iota, _E), axis=1)
        w0 = 1.0 / (1.0 + jnp.exp(m1 - m0))
        idx_ref[...] = jnp.concatenate([i0[:, None], i1[:, None]], axis=1)
        wt_ref[...] = jnp.concatenate([w0, 1.0 - w0], axis=1)

    return pl.pallas_call(
        body,
        out_shape=(jax.ShapeDtypeStruct((N, _K), jnp.int32),
                   jax.ShapeDtypeStruct((N, _K), jnp.float32)),
    )(x, wrt, bias)


def _plan(idx, wt, N):
    P = _NB * _BLK
    e_flat = idx.reshape(-1)
    oh = (e_flat[:, None] == jnp.arange(_E, dtype=jnp.int32)[None, :]).astype(jnp.int32)
    cum = jnp.cumsum(oh, axis=0)
    counts = cum[-1]
    rank = jnp.take_along_axis(cum, e_flat[:, None], axis=1)[:, 0] - 1
    blocks_e = (counts + _BLK - 1) // _BLK
    ends = jnp.cumsum(blocks_e * _BLK).astype(jnp.int32)
    off = jnp.concatenate([jnp.zeros(1, jnp.int32), ends[:-1]])
    pos = off[e_flat] + rank
    a2n = jnp.arange(_K * N, dtype=jnp.int32)
    row_token = jnp.zeros((P,), jnp.int32).at[pos].set(a2n // _K)
    row_gate = jnp.zeros((P, 1), jnp.float32).at[pos, 0].set(wt.reshape(-1))
    bstart = jnp.arange(_NB, dtype=jnp.int32) * _BLK
    block_expert = jnp.minimum(
        jnp.searchsorted(ends, bstart, side="right"), _E - 1).astype(jnp.int32)
    pos0 = pos[0::2].astype(jnp.int32)
    pos1 = pos[1::2].astype(jnp.int32)
    return block_expert, row_token, row_gate, pos0, pos1


def _sc_gather_rows(tok, x, P, H):
    per_w = P // _NW
    ch = 64
    n_ch = per_w // ch
    mesh = plsc.VectorSubcoreMesh(core_axis_name="c", subcore_axis_name="s")

    @functools.partial(
        pl.kernel, mesh=mesh,
        out_type=jax.ShapeDtypeStruct((P, H), jnp.float32),
        scratch_types=[pltpu.VMEM((ch,), jnp.int32),
                       pltpu.VMEM((ch, H), jnp.float32),
                       pltpu.SemaphoreType.DMA],
    )
    def k(tok_hbm, x_hbm, out_hbm, idx_v, rows_v, sem):
        wid = lax.axis_index("s") * 2 + lax.axis_index("c")
        for c in range(n_ch):
            base = wid * per_w + c * ch
            pltpu.sync_copy(tok_hbm.at[pl.ds(base, ch)], idx_v)
            pltpu.async_copy(x_hbm.at[idx_v], rows_v, sem).wait()
            pltpu.sync_copy(rows_v, out_hbm.at[pl.ds(base, ch)])

    return k(tok, x)


def _sc_combine(ao, pos0, pos1, N, H):
    per_w = N // _NW
    ch = 32
    n_ch = per_w // ch
    mesh = plsc.VectorSubcoreMesh(core_axis_name="c", subcore_axis_name="s")

    @functools.partial(
        pl.kernel, mesh=mesh,
        out_type=jax.ShapeDtypeStruct((N, H), jnp.float32),
        scratch_types=[pltpu.VMEM((ch,), jnp.int32),
                       pltpu.VMEM((ch,), jnp.int32),
                       pltpu.VMEM((ch, H), jnp.float32),
                       pltpu.VMEM((ch, H), jnp.float32),
                       pltpu.SemaphoreType.DMA],
    )
    def k(p0_hbm, p1_hbm, ao_hbm, out_hbm, i0_v, i1_v, r0_v, r1_v, sem):
        wid = lax.axis_index("s") * 2 + lax.axis_index("c")
        for c in range(n_ch):
            base = wid * per_w + c * ch
            pltpu.sync_copy(p0_hbm.at[pl.ds(base, ch)], i0_v)
            pltpu.sync_copy(p1_hbm.at[pl.ds(base, ch)], i1_v)
            cp0 = pltpu.async_copy(ao_hbm.at[i0_v], r0_v, sem)
            cp1 = pltpu.async_copy(ao_hbm.at[i1_v], r1_v, sem)
            cp0.wait()
            cp1.wait()

            def row_add(r, carry):
                for j in range(H // 16):
                    sl = pl.ds(j * 16, 16)
                    r0_v[r, sl] = r0_v[r, sl] + r1_v[r, sl]
                return carry

            lax.fori_loop(0, ch, row_add, 0)
            pltpu.sync_copy(r0_v, out_hbm.at[pl.ds(base, ch)])

    return k(pos0, pos1, ao)


def _grouped_ffn(xs, gate, be, w1, b1, w2, b2, wo, bo):
    P, H = xs.shape
    I = w1.shape[1]

    def body(be_ref, xs_ref, g_ref, w1_ref, b1_ref, w2_ref, b2_ref,
             wo_ref, bo_ref, ao_ref):
        xb = xs_ref[...]
        h1 = lax.dot_general(xb, w1_ref[0], (((1,), (1,)), ((), ())),
                             preferred_element_type=jnp.float32) + b1_ref[0]
        h2 = lax.dot_general(xb, w2_ref[0], (((1,), (1,)), ((), ())),
                             preferred_element_type=jnp.float32) + b2_ref[0]
        a = h1 * (1.0 / (1.0 + jnp.exp(-h2)))
        o = lax.dot_general(a, wo_ref[0], (((1,), (1,)), ((), ())),
                            preferred_element_type=jnp.float32) + bo_ref[0]
        ao_ref[...] = o * g_ref[...]

    grid_spec = pltpu.PrefetchScalarGridSpec(
        num_scalar_prefetch=1,
        grid=(_NB,),
        in_specs=[
            pl.BlockSpec((_BLK, H), lambda nb, be: (nb, 0)),
            pl.BlockSpec((_BLK, 1), lambda nb, be: (nb, 0)),
            pl.BlockSpec((1, I, H), lambda nb, be: (be[nb], 0, 0)),
            pl.BlockSpec((1, 1, I), lambda nb, be: (be[nb], 0, 0)),
            pl.BlockSpec((1, I, H), lambda nb, be: (be[nb], 0, 0)),
            pl.BlockSpec((1, 1, I), lambda nb, be: (be[nb], 0, 0)),
            pl.BlockSpec((1, H, I), lambda nb, be: (be[nb], 0, 0)),
            pl.BlockSpec((1, 1, H), lambda nb, be: (be[nb], 0, 0)),
        ],
        out_specs=pl.BlockSpec((_BLK, H), lambda nb, be: (nb, 0)),
    )
    return pl.pallas_call(
        body,
        grid_spec=grid_spec,
        out_shape=jax.ShapeDtypeStruct((P, H), jnp.float32),
    )(be, xs, gate, w1, b1, w2, b2, wo, bo)


def kernel(x, Wr, br, W_in, b_in, W_out, b_out):
    N, H = x.shape
    I = W_out.shape[2]
    u = jax.random.uniform(jax.random.key(42), (N, _E), dtype=jnp.float32)
    noise = -jnp.log(-jnp.log(u + 1e-20) + 1e-20)
    bias = br[None, :] + noise
    idx, wt = _router(x, Wr.T, bias)
    be, row_token, row_gate, pos0, pos1 = _plan(idx, wt, N)
    xs = _sc_gather_rows(row_token, x, _NB * _BLK, H)
    w1 = W_in[:, :I, :]
    w2 = W_in[:, I:, :]
    b1 = b_in[:, :I].reshape(_E, 1, I)
    b2 = b_in[:, I:].reshape(_E, 1, I)
    bo = b_out.reshape(_E, 1, H)
    ao = _grouped_ffn(xs, row_gate, be, w1, b1, w2, b2, W_out, bo)
    return _sc_combine(ao, pos0, pos1, N, H)


# speedup vs baseline: 1.2155x; 1.2155x over previous
import functools

import jax
import jax.numpy as jnp
from jax import lax
from jax.experimental import pallas as pl
from jax.experimental.pallas import tpu as pltpu
from jax.experimental.pallas import tpu_sc as plsc

_E = 8
_K = 2
_BLK = 256
_NB = 24
_NW = 32


def _router(x, wrt, bias):
    N, _ = x.shape

    def body(x_ref, w_ref, b_ref, idx_ref, wt_ref):
        y = lax.dot_general(x_ref[...], w_ref[...], (((1,), (0,)), ((), ())),
                            preferred_element_type=jnp.float32) + b_ref[...]
        iota = lax.broadcasted_iota(jnp.int32, y.shape, 1)
        m0 = jnp.max(y, axis=1, keepdims=True)
        i0 = jnp.min(jnp.where(y == m0, iota, _E), axis=1)
        ymask = jnp.where(iota == i0[:, None], -1e30, y)
        m1 = jnp.max(ymask, axis=1, keepdims=True)
        i1 = jnp.min(jnp.where(ymask == m1, iota, _E), axis=1)
        w0 = 1.0 / (1.0 + jnp.exp(m1 - m0))
        idx_ref[...] = jnp.concatenate([i0[:, None], i1[:, None]], axis=1)
        wt_ref[...] = jnp.concatenate([w0, 1.0 - w0], axis=1)

    return pl.pallas_call(
        body,
        out_shape=(jax.ShapeDtypeStruct((N, _K), jnp.int32),
                   jax.ShapeDtypeStruct((N, _K), jnp.float32)),
    )(x, wrt, bias)


def _plan(idx, wt, N):
    P = _NB * _BLK
    e_flat = idx.reshape(-1)
    oh = (e_flat[:, None] == jnp.arange(_E, dtype=jnp.int32)[None, :]).astype(jnp.int32)
    cum = jnp.cumsum(oh, axis=0)
    counts = cum[-1]
    rank = jnp.take_along_axis(cum, e_flat[:, None], axis=1)[:, 0] - 1
    blocks_e = (counts + _BLK - 1) // _BLK
    ends = jnp.cumsum(blocks_e * _BLK).astype(jnp.int32)
    off = jnp.concatenate([jnp.zeros(1, jnp.int32), ends[:-1]])
    pos = off[e_flat] + rank
    a2n = jnp.arange(_K * N, dtype=jnp.int32)
    pad_tok = jnp.arange(P, dtype=jnp.int32) % N
    row_token = pad_tok.at[pos].set(a2n // _K)
    row_gate = jnp.zeros((P, 1), jnp.float32).at[pos, 0].set(wt.reshape(-1))
    bstart = jnp.arange(_NB, dtype=jnp.int32) * _BLK
    block_expert = jnp.minimum(
        jnp.searchsorted(ends, bstart, side="right"), _E - 1).astype(jnp.int32)
    pos0 = pos[0::2].astype(jnp.int32)
    pos1 = pos[1::2].astype(jnp.int32)
    return block_expert, row_token, row_gate, pos0, pos1


def _sc_gather_rows(tok, x, P, H):
    per_w = P // _NW
    ch = 64
    n_ch = per_w // ch
    mesh = plsc.VectorSubcoreMesh(core_axis_name="c", subcore_axis_name="s")

    @functools.partial(
        pl.kernel, mesh=mesh,
        out_type=jax.ShapeDtypeStruct((P, H), jnp.float32),
        scratch_types=[pltpu.VMEM((ch,), jnp.int32),
                       pltpu.VMEM((ch, H), jnp.float32),
                       pltpu.SemaphoreType.DMA],
    )
    def k(tok_hbm, x_hbm, out_hbm, idx_v, rows_v, sem):
        wid = lax.axis_index("s") * 2 + lax.axis_index("c")
        for c in range(n_ch):
            base = wid * per_w + c * ch
            pltpu.sync_copy(tok_hbm.at[pl.ds(base, ch)], idx_v)
            pltpu.async_copy(x_hbm.at[idx_v], rows_v, sem).wait()
            pltpu.sync_copy(rows_v, out_hbm.at[pl.ds(base, ch)])

    return k(tok, x)


def _sc_combine(ao, pos0, pos1, N, H):
    per_w = N // _NW
    ch = 32
    n_ch = per_w // ch
    mesh = plsc.VectorSubcoreMesh(core_axis_name="c", subcore_axis_name="s")

    @functools.partial(
        pl.kernel, mesh=mesh,
        out_type=jax.ShapeDtypeStruct((N, H), jnp.float32),
        scratch_types=[pltpu.VMEM((ch,), jnp.int32),
                       pltpu.VMEM((ch,), jnp.int32),
                       pltpu.VMEM((ch, H), jnp.float32),
                       pltpu.VMEM((ch, H), jnp.float32),
                       pltpu.SemaphoreType.DMA],
    )
    def k(p0_hbm, p1_hbm, ao_hbm, out_hbm, i0_v, i1_v, r0_v, r1_v, sem):
        wid = lax.axis_index("s") * 2 + lax.axis_index("c")
        for c in range(n_ch):
            base = wid * per_w + c * ch
            pltpu.sync_copy(p0_hbm.at[pl.ds(base, ch)], i0_v)
            pltpu.sync_copy(p1_hbm.at[pl.ds(base, ch)], i1_v)
            cp0 = pltpu.async_copy(ao_hbm.at[i0_v], r0_v, sem)
            cp1 = pltpu.async_copy(ao_hbm.at[i1_v], r1_v, sem)
            cp0.wait()
            cp1.wait()

            def row_add(r, carry):
                for j in range(H // 16):
                    sl = pl.ds(j * 16, 16)
                    r0_v[r, sl] = r0_v[r, sl] + r1_v[r, sl]
                return carry

            lax.fori_loop(0, ch, row_add, 0)
            pltpu.sync_copy(r0_v, out_hbm.at[pl.ds(base, ch)])

    return k(pos0, pos1, ao)


def _grouped_ffn(xs, gate, be, w1, b1, w2, b2, wo, bo):
    P, H = xs.shape
    I = w1.shape[1]

    def body(be_ref, xs_ref, g_ref, w1_ref, b1_ref, w2_ref, b2_ref,
             wo_ref, bo_ref, ao_ref):
        xb = xs_ref[...].astype(jnp.bfloat16)
        h1 = lax.dot_general(xb, w1_ref[0], (((1,), (1,)), ((), ())),
                             preferred_element_type=jnp.float32) + b1_ref[0]
        h2 = lax.dot_general(xb, w2_ref[0], (((1,), (1,)), ((), ())),
                             preferred_element_type=jnp.float32) + b2_ref[0]
        a = (h1 * (1.0 / (1.0 + jnp.exp(-h2)))).astype(jnp.bfloat16)
        o = lax.dot_general(a, wo_ref[0], (((1,), (1,)), ((), ())),
                            preferred_element_type=jnp.float32) + bo_ref[0]
        ao_ref[...] = o * g_ref[...]

    grid_spec = pltpu.PrefetchScalarGridSpec(
        num_scalar_prefetch=1,
        grid=(_NB,),
        in_specs=[
            pl.BlockSpec((_BLK, H), lambda nb, be: (nb, 0)),
            pl.BlockSpec((_BLK, 1), lambda nb, be: (nb, 0)),
            pl.BlockSpec((1, I, H), lambda nb, be: (be[nb], 0, 0)),
            pl.BlockSpec((1, 1, I), lambda nb, be: (be[nb], 0, 0)),
            pl.BlockSpec((1, I, H), lambda nb, be: (be[nb], 0, 0)),
            pl.BlockSpec((1, 1, I), lambda nb, be: (be[nb], 0, 0)),
            pl.BlockSpec((1, H, I), lambda nb, be: (be[nb], 0, 0)),
            pl.BlockSpec((1, 1, H), lambda nb, be: (be[nb], 0, 0)),
        ],
        out_specs=pl.BlockSpec((_BLK, H), lambda nb, be: (nb, 0)),
    )
    return pl.pallas_call(
        body,
        grid_spec=grid_spec,
        out_shape=jax.ShapeDtypeStruct((P, H), jnp.float32),
    )(be, xs, gate, w1, b1, w2, b2, wo, bo)


def kernel(x, Wr, br, W_in, b_in, W_out, b_out):
    N, H = x.shape
    I = W_out.shape[2]
    u = jax.random.uniform(jax.random.key(42), (N, _E), dtype=jnp.float32)
    noise = -jnp.log(-jnp.log(u + 1e-20) + 1e-20)
    bias = br[None, :] + noise
    idx, wt = _router(x, Wr.T, bias)
    be, row_token, row_gate, pos0, pos1 = _plan(idx, wt, N)
    xs = _sc_gather_rows(row_token, x, _NB * _BLK, H)
    w16 = W_in.astype(jnp.bfloat16)
    w1 = w16[:, :I, :]
    w2 = w16[:, I:, :]
    b1 = b_in[:, :I].reshape(_E, 1, I)
    b2 = b_in[:, I:].reshape(_E, 1, I)
    bo = b_out.reshape(_E, 1, H)
    ao = _grouped_ffn(xs, row_gate, be, w1, b1, w2, b2,
                      W_out.astype(jnp.bfloat16), bo)
    return _sc_combine(ao, pos0, pos1, N, H)


# speedup vs baseline: 1.2918x; 1.0628x over previous
import functools

import jax
import jax.numpy as jnp
from jax import lax
from jax.experimental import pallas as pl
from jax.experimental.pallas import tpu as pltpu
from jax.experimental.pallas import tpu_sc as plsc

_E = 8
_K = 2
_BLK = 256
_NB = 24
_NW = 32


def _router(x, wrt, bias):
    N, _ = x.shape

    def body(x_ref, w_ref, b_ref, idx_ref, wt_ref):
        y = lax.dot_general(x_ref[...], w_ref[...], (((1,), (0,)), ((), ())),
                            preferred_element_type=jnp.float32) + b_ref[...]
        iota = lax.broadcasted_iota(jnp.int32, y.shape, 1)
        m0 = jnp.max(y, axis=1, keepdims=True)
        i0 = jnp.min(jnp.where(y == m0, iota, _E), axis=1)
        ymask = jnp.where(iota == i0[:, None], -1e30, y)
        m1 = jnp.max(ymask, axis=1, keepdims=True)
        i1 = jnp.min(jnp.where(ymask == m1, iota, _E), axis=1)
        w0 = 1.0 / (1.0 + jnp.exp(m1 - m0))
        idx_ref[...] = jnp.concatenate([i0[:, None], i1[:, None]], axis=1)
        wt_ref[...] = jnp.concatenate([w0, 1.0 - w0], axis=1)

    return pl.pallas_call(
        body,
        out_shape=(jax.ShapeDtypeStruct((N, _K), jnp.int32),
                   jax.ShapeDtypeStruct((N, _K), jnp.float32)),
    )(x, wrt, bias)


def _plan(idx, wt, N):
    P = _NB * _BLK
    e_flat = idx.reshape(-1)
    oh = (e_flat[:, None] == jnp.arange(_E, dtype=jnp.int32)[None, :]).astype(jnp.int32)
    cum = jnp.cumsum(oh, axis=0)
    counts = cum[-1]
    rank = jnp.take_along_axis(cum, e_flat[:, None], axis=1)[:, 0] - 1
    blocks_e = (counts + _BLK - 1) // _BLK
    ends = jnp.cumsum(blocks_e * _BLK).astype(jnp.int32)
    off = jnp.concatenate([jnp.zeros(1, jnp.int32), ends[:-1]])
    pos = off[e_flat] + rank
    a2n = jnp.arange(_K * N, dtype=jnp.int32)
    pad_tok = jnp.arange(P, dtype=jnp.int32) % N
    row_token = pad_tok.at[pos].set(a2n // _K)
    row_gate = jnp.zeros((P, 1), jnp.float32).at[pos, 0].set(wt.reshape(-1))
    bstart = jnp.arange(_NB, dtype=jnp.int32) * _BLK
    block_expert = jnp.minimum(
        jnp.searchsorted(ends, bstart, side="right"), _E - 1).astype(jnp.int32)
    pos0 = pos[0::2].astype(jnp.int32)
    pos1 = pos[1::2].astype(jnp.int32)
    return block_expert, row_token, row_gate, pos0, pos1


def _sc_gather_rows(tok, x, P, H):
    per_w = P // _NW
    ch = 64
    n_ch = per_w // ch
    mesh = plsc.VectorSubcoreMesh(core_axis_name="c", subcore_axis_name="s")

    @functools.partial(
        pl.kernel, mesh=mesh,
        out_type=jax.ShapeDtypeStruct((P, H), jnp.float32),
        scratch_types=[pltpu.VMEM((ch,), jnp.int32),
                       pltpu.VMEM((ch, H), jnp.float32),
                       pltpu.SemaphoreType.DMA],
    )
    def k(tok_hbm, x_hbm, out_hbm, idx_v, rows_v, sem):
        wid = lax.axis_index("s") * 2 + lax.axis_index("c")
        for c in range(n_ch):
            base = wid * per_w + c * ch
            pltpu.sync_copy(tok_hbm.at[pl.ds(base, ch)], idx_v)
            pltpu.async_copy(x_hbm.at[idx_v], rows_v, sem).wait()
            pltpu.sync_copy(rows_v, out_hbm.at[pl.ds(base, ch)])

    return k(tok, x)


def _sc_combine(ao, pos0, pos1, N, H):
    per_w = N // _NW
    ch = 32
    n_ch = per_w // ch
    mesh = plsc.VectorSubcoreMesh(core_axis_name="c", subcore_axis_name="s")

    @functools.partial(
        pl.kernel, mesh=mesh,
        out_type=jax.ShapeDtypeStruct((N, H), jnp.float32),
        scratch_types=[pltpu.VMEM((ch,), jnp.int32),
                       pltpu.VMEM((ch,), jnp.int32),
                       pltpu.VMEM((ch, H), jnp.float32),
                       pltpu.VMEM((ch, H), jnp.float32),
                       pltpu.SemaphoreType.DMA],
    )
    def k(p0_hbm, p1_hbm, ao_hbm, out_hbm, i0_v, i1_v, r0_v, r1_v, sem):
        wid = lax.axis_index("s") * 2 + lax.axis_index("c")
        for c in range(n_ch):
            base = wid * per_w + c * ch
            pltpu.sync_copy(p0_hbm.at[pl.ds(base, ch)], i0_v)
            pltpu.sync_copy(p1_hbm.at[pl.ds(base, ch)], i1_v)
            cp0 = pltpu.async_copy(ao_hbm.at[i0_v], r0_v, sem)
            cp1 = pltpu.async_copy(ao_hbm.at[i1_v], r1_v, sem)
            cp0.wait()
            cp1.wait()

            def row_add(r, carry):
                for j in range(H // 16):
                    sl = pl.ds(j * 16, 16)
                    r0_v[r, sl] = r0_v[r, sl] + r1_v[r, sl]
                return carry

            lax.fori_loop(0, ch, row_add, 0)
            pltpu.sync_copy(r0_v, out_hbm.at[pl.ds(base, ch)])

    return k(pos0, pos1, ao)


def _grouped_ffn(xs, gate, be, w1, b1, w2, b2, wo, bo):
    P, H = xs.shape
    I = w1.shape[1]

    def body(be_ref, xs_ref, g_ref, w1_ref, b1_ref, w2_ref, b2_ref,
             wo_ref, bo_ref, ao_ref):
        xb = xs_ref[...].astype(jnp.bfloat16)
        w1b = w1_ref[0].astype(jnp.bfloat16)
        w2b = w2_ref[0].astype(jnp.bfloat16)
        wob = wo_ref[0].astype(jnp.bfloat16)
        h1 = lax.dot_general(xb, w1b, (((1,), (1,)), ((), ())),
                             preferred_element_type=jnp.float32) + b1_ref[0]
        h2 = lax.dot_general(xb, w2b, (((1,), (1,)), ((), ())),
                             preferred_element_type=jnp.float32) + b2_ref[0]
        a = (h1 * (1.0 / (1.0 + jnp.exp(-h2)))).astype(jnp.bfloat16)
        o = lax.dot_general(a, wob, (((1,), (1,)), ((), ())),
                            preferred_element_type=jnp.float32) + bo_ref[0]
        ao_ref[...] = o * g_ref[...]

    grid_spec = pltpu.PrefetchScalarGridSpec(
        num_scalar_prefetch=1,
        grid=(_NB,),
        in_specs=[
            pl.BlockSpec((_BLK, H), lambda nb, be: (nb, 0)),
            pl.BlockSpec((_BLK, 1), lambda nb, be: (nb, 0)),
            pl.BlockSpec((1, I, H), lambda nb, be: (be[nb], 0, 0)),
            pl.BlockSpec((1, 1, I), lambda nb, be: (be[nb], 0, 0)),
            pl.BlockSpec((1, I, H), lambda nb, be: (be[nb], 0, 0)),
            pl.BlockSpec((1, 1, I), lambda nb, be: (be[nb], 0, 0)),
            pl.BlockSpec((1, H, I), lambda nb, be: (be[nb], 0, 0)),
            pl.BlockSpec((1, 1, H), lambda nb, be: (be[nb], 0, 0)),
        ],
        out_specs=pl.BlockSpec((_BLK, H), lambda nb, be: (nb, 0)),
    )
    return pl.pallas_call(
        body,
        grid_spec=grid_spec,
        out_shape=jax.ShapeDtypeStruct((P, H), jnp.float32),
    )(be, xs, gate, w1, b1, w2, b2, wo, bo)


def kernel(x, Wr, br, W_in, b_in, W_out, b_out):
    N, H = x.shape
    I = W_out.shape[2]
    u = jax.random.uniform(jax.random.key(42), (N, _E), dtype=jnp.float32)
    noise = -jnp.log(-jnp.log(u + 1e-20) + 1e-20)
    bias = br[None, :] + noise
    idx, wt = _router(x, Wr.T, bias)
    be, row_token, row_gate, pos0, pos1 = _plan(idx, wt, N)
    xs = _sc_gather_rows(row_token, x, _NB * _BLK, H)
    w1 = W_in[:, :I, :]
    w2 = W_in[:, I:, :]
    b1 = b_in[:, :I].reshape(_E, 1, I)
    b2 = b_in[:, I:].reshape(_E, 1, I)
    bo = b_out.reshape(_E, 1, H)
    ao = _grouped_ffn(xs, row_gate, be, w1, b1, w2, b2, W_out, bo)
    return _sc_combine(ao, pos0, pos1, N, H)


# speedup vs baseline: 1.3940x; 1.0791x over previous
import functools

import jax
import jax.numpy as jnp
from jax import lax
from jax.experimental import pallas as pl
from jax.experimental.pallas import tpu as pltpu
from jax.experimental.pallas import tpu_sc as plsc

_E = 8
_K = 2
_BLK = 256
_NB = 24
_NW = 32


def _router(x, wrt, bias):
    N, _ = x.shape

    def body(x_ref, w_ref, b_ref, idx_ref, wt_ref):
        y = lax.dot_general(x_ref[...], w_ref[...], (((1,), (0,)), ((), ())),
                            preferred_element_type=jnp.float32) + b_ref[...]
        iota = lax.broadcasted_iota(jnp.int32, y.shape, 1)
        m0 = jnp.max(y, axis=1, keepdims=True)
        i0 = jnp.min(jnp.where(y == m0, iota, _E), axis=1)
        ymask = jnp.where(iota == i0[:, None], -1e30, y)
        m1 = jnp.max(ymask, axis=1, keepdims=True)
        i1 = jnp.min(jnp.where(ymask == m1, iota, _E), axis=1)
        w0 = 1.0 / (1.0 + jnp.exp(m1 - m0))
        idx_ref[...] = jnp.concatenate([i0[:, None], i1[:, None]], axis=1)
        wt_ref[...] = jnp.concatenate([w0, 1.0 - w0], axis=1)

    return pl.pallas_call(
        body,
        out_shape=(jax.ShapeDtypeStruct((N, _K), jnp.int32),
                   jax.ShapeDtypeStruct((N, _K), jnp.float32)),
    )(x, wrt, bias)


def _plan(idx, wt, N):
    P = _NB * _BLK
    M = _K * N
    C = 128
    e_flat = idx.reshape(-1)
    oh = (e_flat[:, None] == jnp.arange(_E, dtype=jnp.int32)[None, :]).astype(jnp.float32)
    oh3 = oh.reshape(M // C, C, _E)
    tri = (jnp.arange(C)[:, None] >= jnp.arange(C)[None, :]).astype(jnp.float32)
    within = jnp.einsum('rc,bce->bre', tri, oh3,
                        preferred_element_type=jnp.float32)
    chunk_tot = oh3.sum(axis=1)
    tri2 = (jnp.arange(M // C)[:, None] > jnp.arange(M // C)[None, :]).astype(jnp.float32)
    carry = tri2 @ chunk_tot
    cum = (within + carry[:, None, :]).reshape(M, _E)
    counts = chunk_tot.sum(axis=0).astype(jnp.int32)
    rank = (oh * cum).sum(axis=1).astype(jnp.int32) - 1
    blocks_e = (counts + _BLK - 1) // _BLK
    ends = jnp.cumsum(blocks_e * _BLK).astype(jnp.int32)
    off = jnp.concatenate([jnp.zeros(1, jnp.int32), ends[:-1]])
    pos = (oh * off[None, :].astype(jnp.float32)).sum(axis=1).astype(jnp.int32) + rank
    a2n = jnp.arange(_K * N, dtype=jnp.int32)
    pad_tok = jnp.arange(P, dtype=jnp.int32) % N
    row_token = pad_tok.at[pos].set(a2n // _K)
    row_gate = jnp.zeros((P, 1), jnp.float32).at[pos, 0].set(wt.reshape(-1))
    bstart = jnp.arange(_NB, dtype=jnp.int32) * _BLK
    block_expert = jnp.minimum(
        (bstart[:, None] >= ends[None, :]).astype(jnp.int32).sum(axis=1),
        _E - 1).astype(jnp.int32)
    pos0 = pos[0::2].astype(jnp.int32)
    pos1 = pos[1::2].astype(jnp.int32)
    return block_expert, row_token, row_gate, pos0, pos1


def _sc_gather_rows(tok, x, P, H):
    per_w = P // _NW
    ch = 64
    n_ch = per_w // ch
    mesh = plsc.VectorSubcoreMesh(core_axis_name="c", subcore_axis_name="s")

    @functools.partial(
        pl.kernel, mesh=mesh,
        out_type=jax.ShapeDtypeStruct((P, H), jnp.float32),
        scratch_types=[pltpu.VMEM((ch,), jnp.int32),
                       pltpu.VMEM((ch, H), jnp.float32),
                       pltpu.SemaphoreType.DMA],
    )
    def k(tok_hbm, x_hbm, out_hbm, idx_v, rows_v, sem):
        wid = lax.axis_index("s") * 2 + lax.axis_index("c")
        for c in range(n_ch):
            base = wid * per_w + c * ch
            pltpu.sync_copy(tok_hbm.at[pl.ds(base, ch)], idx_v)
            pltpu.async_copy(x_hbm.at[idx_v], rows_v, sem).wait()
            pltpu.sync_copy(rows_v, out_hbm.at[pl.ds(base, ch)])

    return k(tok, x)


def _sc_combine(ao, pos0, pos1, N, H):
    per_w = N // _NW
    ch = 32
    n_ch = per_w // ch
    mesh = plsc.VectorSubcoreMesh(core_axis_name="c", subcore_axis_name="s")

    @functools.partial(
        pl.kernel, mesh=mesh,
        out_type=jax.ShapeDtypeStruct((N, H), jnp.float32),
        scratch_types=[pltpu.VMEM((ch,), jnp.int32),
                       pltpu.VMEM((ch,), jnp.int32),
                       pltpu.VMEM((ch, H), jnp.float32),
                       pltpu.VMEM((ch, H), jnp.float32),
                       pltpu.SemaphoreType.DMA],
    )
    def k(p0_hbm, p1_hbm, ao_hbm, out_hbm, i0_v, i1_v, r0_v, r1_v, sem):
        wid = lax.axis_index("s") * 2 + lax.axis_index("c")
        for c in range(n_ch):
            base = wid * per_w + c * ch
            pltpu.sync_copy(p0_hbm.at[pl.ds(base, ch)], i0_v)
            pltpu.sync_copy(p1_hbm.at[pl.ds(base, ch)], i1_v)
            cp0 = pltpu.async_copy(ao_hbm.at[i0_v], r0_v, sem)
            cp1 = pltpu.async_copy(ao_hbm.at[i1_v], r1_v, sem)
            cp0.wait()
            cp1.wait()

            def row_add(r, carry):
                for j in range(H // 16):
                    sl = pl.ds(j * 16, 16)
                    r0_v[r, sl] = r0_v[r, sl] + r1_v[r, sl]
                return carry

            lax.fori_loop(0, ch, row_add, 0)
            pltpu.sync_copy(r0_v, out_hbm.at[pl.ds(base, ch)])

    return k(pos0, pos1, ao)


def _grouped_ffn(xs, gate, be, w1, b1, w2, b2, wo, bo):
    P, H = xs.shape
    I = w1.shape[1]

    def body(be_ref, xs_ref, g_ref, w1_ref, b1_ref, w2_ref, b2_ref,
             wo_ref, bo_ref, ao_ref):
        xb = xs_ref[...].astype(jnp.bfloat16)
        w1b = w1_ref[0].astype(jnp.bfloat16)
        w2b = w2_ref[0].astype(jnp.bfloat16)
        wob = wo_ref[0].astype(jnp.bfloat16)
        h1 = lax.dot_general(xb, w1b, (((1,), (1,)), ((), ())),
                             preferred_element_type=jnp.float32) + b1_ref[0]
        h2 = lax.dot_general(xb, w2b, (((1,), (1,)), ((), ())),
                             preferred_element_type=jnp.float32) + b2_ref[0]
        a = (h1 * (1.0 / (1.0 + jnp.exp(-h2)))).astype(jnp.bfloat16)
        o = lax.dot_general(a, wob, (((1,), (1,)), ((), ())),
                            preferred_element_type=jnp.float32) + bo_ref[0]
        ao_ref[...] = o * g_ref[...]

    grid_spec = pltpu.PrefetchScalarGridSpec(
        num_scalar_prefetch=1,
        grid=(_NB,),
        in_specs=[
            pl.BlockSpec((_BLK, H), lambda nb, be: (nb, 0)),
            pl.BlockSpec((_BLK, 1), lambda nb, be: (nb, 0)),
            pl.BlockSpec((1, I, H), lambda nb, be: (be[nb], 0, 0)),
            pl.BlockSpec((1, 1, I), lambda nb, be: (be[nb], 0, 0)),
            pl.BlockSpec((1, I, H), lambda nb, be: (be[nb], 0, 0)),
            pl.BlockSpec((1, 1, I), lambda nb, be: (be[nb], 0, 0)),
            pl.BlockSpec((1, H, I), lambda nb, be: (be[nb], 0, 0)),
            pl.BlockSpec((1, 1, H), lambda nb, be: (be[nb], 0, 0)),
        ],
        out_specs=pl.BlockSpec((_BLK, H), lambda nb, be: (nb, 0)),
    )
    return pl.pallas_call(
        body,
        grid_spec=grid_spec,
        out_shape=jax.ShapeDtypeStruct((P, H), jnp.float32),
    )(be, xs, gate, w1, b1, w2, b2, wo, bo)


def kernel(x, Wr, br, W_in, b_in, W_out, b_out):
    N, H = x.shape
    I = W_out.shape[2]
    u = jax.random.uniform(jax.random.key(42), (N, _E), dtype=jnp.float32)
    noise = -jnp.log(-jnp.log(u + 1e-20) + 1e-20)
    bias = br[None, :] + noise
    idx, wt = _router(x, Wr.T, bias)
    be, row_token, row_gate, pos0, pos1 = _plan(idx, wt, N)
    xs = _sc_gather_rows(row_token, x, _NB * _BLK, H)
    w1 = W_in[:, :I, :]
    w2 = W_in[:, I:, :]
    b1 = b_in[:, :I].reshape(_E, 1, I)
    b2 = b_in[:, I:].reshape(_E, 1, I)
    bo = b_out.reshape(_E, 1, H)
    ao = _grouped_ffn(xs, row_gate, be, w1, b1, w2, b2, W_out, bo)
    return _sc_combine(ao, pos0, pos1, N, H)


# speedup vs baseline: 1.3995x; 1.0040x over previous
import functools

import jax
import jax.numpy as jnp
from jax import lax
from jax.experimental import pallas as pl
from jax.experimental.pallas import tpu as pltpu
from jax.experimental.pallas import tpu_sc as plsc

_E = 8
_K = 2
_BLK = 256
_NB = 24
_NW = 32


def _router(x, wrt, bias):
    N, _ = x.shape

    def body(x_ref, w_ref, b_ref, idx_ref, wt_ref):
        y = lax.dot_general(x_ref[...], w_ref[...], (((1,), (0,)), ((), ())),
                            preferred_element_type=jnp.float32) + b_ref[...]
        iota = lax.broadcasted_iota(jnp.int32, y.shape, 1)
        m0 = jnp.max(y, axis=1, keepdims=True)
        i0 = jnp.min(jnp.where(y == m0, iota, _E), axis=1)
        ymask = jnp.where(iota == i0[:, None], -1e30, y)
        m1 = jnp.max(ymask, axis=1, keepdims=True)
        i1 = jnp.min(jnp.where(ymask == m1, iota, _E), axis=1)
        w0 = 1.0 / (1.0 + jnp.exp(m1 - m0))
        idx_ref[...] = jnp.concatenate([i0[:, None], i1[:, None]], axis=1)
        wt_ref[...] = jnp.concatenate([w0, 1.0 - w0], axis=1)

    return pl.pallas_call(
        body,
        out_shape=(jax.ShapeDtypeStruct((N, _K), jnp.int32),
                   jax.ShapeDtypeStruct((N, _K), jnp.float32)),
    )(x, wrt, bias)


def _plan(idx, wt, N):
    P = _NB * _BLK
    M = _K * N
    C = 128
    e_flat = idx.reshape(-1)
    oh = (e_flat[:, None] == jnp.arange(_E, dtype=jnp.int32)[None, :]).astype(jnp.float32)
    oh3 = oh.reshape(M // C, C, _E)
    tri = (jnp.arange(C)[:, None] >= jnp.arange(C)[None, :]).astype(jnp.float32)
    within = jnp.einsum('rc,bce->bre', tri, oh3,
                        preferred_element_type=jnp.float32)
    chunk_tot = oh3.sum(axis=1)
    tri2 = (jnp.arange(M // C)[:, None] > jnp.arange(M // C)[None, :]).astype(jnp.float32)
    carry = tri2 @ chunk_tot
    cum = (within + carry[:, None, :]).reshape(M, _E)
    counts = chunk_tot.sum(axis=0).astype(jnp.int32)
    rank = (oh * cum).sum(axis=1).astype(jnp.int32) - 1
    blocks_e = (counts + _BLK - 1) // _BLK
    ends = jnp.cumsum(blocks_e * _BLK).astype(jnp.int32)
    off = jnp.concatenate([jnp.zeros(1, jnp.int32), ends[:-1]])
    pos = (oh * off[None, :].astype(jnp.float32)).sum(axis=1).astype(jnp.int32) + rank
    a2n = jnp.arange(_K * N, dtype=jnp.int32)
    pad_tok = jnp.arange(P, dtype=jnp.int32) % N
    row_token = pad_tok.at[pos].set(a2n // _K)
    row_gate = jnp.zeros((P, 1), jnp.float32).at[pos, 0].set(wt.reshape(-1))
    bstart = jnp.arange(_NB, dtype=jnp.int32) * _BLK
    block_expert = jnp.minimum(
        (bstart[:, None] >= ends[None, :]).astype(jnp.int32).sum(axis=1),
        _E - 1).astype(jnp.int32)
    pos0 = pos[0::2].astype(jnp.int32)
    pos1 = pos[1::2].astype(jnp.int32)
    return block_expert, row_token, row_gate, pos0, pos1


def _sc_gather_rows(tok, x, P, H):
    per_w = P // _NW
    ch = 64
    n_ch = per_w // ch
    mesh = plsc.VectorSubcoreMesh(core_axis_name="c", subcore_axis_name="s")

    @functools.partial(
        pl.kernel, mesh=mesh,
        out_type=jax.ShapeDtypeStruct((P, H), jnp.float32),
        scratch_types=[pltpu.VMEM((ch,), jnp.int32),
                       pltpu.VMEM((ch, H), jnp.float32),
                       pltpu.SemaphoreType.DMA],
    )
    def k(tok_hbm, x_hbm, out_hbm, idx_v, rows_v, sem):
        wid = lax.axis_index("s") * 2 + lax.axis_index("c")
        for c in range(n_ch):
            base = wid * per_w + c * ch
            pltpu.sync_copy(tok_hbm.at[pl.ds(base, ch)], idx_v)
            pltpu.async_copy(x_hbm.at[idx_v], rows_v, sem).wait()
            pltpu.sync_copy(rows_v, out_hbm.at[pl.ds(base, ch)])

    return k(tok, x)


def _sc_combine(ao, pos0, pos1, N, H):
    per_w = N // _NW
    ch = 32
    n_ch = per_w // ch
    mesh = plsc.VectorSubcoreMesh(core_axis_name="c", subcore_axis_name="s")

    @functools.partial(
        pl.kernel, mesh=mesh,
        out_type=jax.ShapeDtypeStruct((N, H), jnp.float32),
        scratch_types=[pltpu.VMEM((ch,), jnp.int32),
                       pltpu.VMEM((ch,), jnp.int32),
                       pltpu.VMEM((ch, H), jnp.float32),
                       pltpu.VMEM((ch, H), jnp.float32),
                       pltpu.SemaphoreType.DMA],
    )
    def k(p0_hbm, p1_hbm, ao_hbm, out_hbm, i0_v, i1_v, r0_v, r1_v, sem):
        wid = lax.axis_index("s") * 2 + lax.axis_index("c")
        for c in range(n_ch):
            base = wid * per_w + c * ch
            pltpu.sync_copy(p0_hbm.at[pl.ds(base, ch)], i0_v)
            pltpu.sync_copy(p1_hbm.at[pl.ds(base, ch)], i1_v)
            cp0 = pltpu.async_copy(ao_hbm.at[i0_v], r0_v, sem)
            cp1 = pltpu.async_copy(ao_hbm.at[i1_v], r1_v, sem)
            cp0.wait()
            cp1.wait()

            def row_add(r, carry):
                for j in range(H // 16):
                    sl = pl.ds(j * 16, 16)
                    r0_v[r, sl] = r0_v[r, sl] + r1_v[r, sl]
                return carry

            lax.fori_loop(0, ch, row_add, 0)
            pltpu.sync_copy(r0_v, out_hbm.at[pl.ds(base, ch)])

    return k(pos0, pos1, ao)


def _grouped_ffn(xs, gate, be, w1, b1, w2, b2, wo, bo):
    P, H = xs.shape
    I = w1.shape[1]

    def body(be_ref, xs_ref, g_ref, w1_ref, b1_ref, w2_ref, b2_ref,
             wo_ref, bo_ref, ao_ref):
        xb = xs_ref[...]
        h1 = lax.dot_general(xb, w1_ref[0], (((1,), (1,)), ((), ())),
                             preferred_element_type=jnp.float32) + b1_ref[0]
        h2 = lax.dot_general(xb, w2_ref[0], (((1,), (1,)), ((), ())),
                             preferred_element_type=jnp.float32) + b2_ref[0]
        a = h1 * (1.0 / (1.0 + jnp.exp(-h2)))
        o = lax.dot_general(a, wo_ref[0], (((1,), (1,)), ((), ())),
                            preferred_element_type=jnp.float32) + bo_ref[0]
        ao_ref[...] = o * g_ref[...]

    grid_spec = pltpu.PrefetchScalarGridSpec(
        num_scalar_prefetch=1,
        grid=(_NB,),
        in_specs=[
            pl.BlockSpec((_BLK, H), lambda nb, be: (nb, 0)),
            pl.BlockSpec((_BLK, 1), lambda nb, be: (nb, 0)),
            pl.BlockSpec((1, I, H), lambda nb, be: (be[nb], 0, 0)),
            pl.BlockSpec((1, 1, I), lambda nb, be: (be[nb], 0, 0)),
            pl.BlockSpec((1, I, H), lambda nb, be: (be[nb], 0, 0)),
            pl.BlockSpec((1, 1, I), lambda nb, be: (be[nb], 0, 0)),
            pl.BlockSpec((1, H, I), lambda nb, be: (be[nb], 0, 0)),
            pl.BlockSpec((1, 1, H), lambda nb, be: (be[nb], 0, 0)),
        ],
        out_specs=pl.BlockSpec((_BLK, H), lambda nb, be: (nb, 0)),
    )
    return pl.pallas_call(
        body,
        grid_spec=grid_spec,
        out_shape=jax.ShapeDtypeStruct((P, H), jnp.float32),
    )(be, xs, gate, w1, b1, w2, b2, wo, bo)


def kernel(x, Wr, br, W_in, b_in, W_out, b_out):
    N, H = x.shape
    I = W_out.shape[2]
    u = jax.random.uniform(jax.random.key(42), (N, _E), dtype=jnp.float32)
    noise = -jnp.log(-jnp.log(u + 1e-20) + 1e-20)
    bias = br[None, :] + noise
    idx, wt = _router(x, Wr.T, bias)
    be, row_token, row_gate, pos0, pos1 = _plan(idx, wt, N)
    xs = _sc_gather_rows(row_token, x, _NB * _BLK, H)
    w1 = W_in[:, :I, :]
    w2 = W_in[:, I:, :]
    b1 = b_in[:, :I].reshape(_E, 1, I)
    b2 = b_in[:, I:].reshape(_E, 1, I)
    bo = b_out.reshape(_E, 1, H)
    ao = _grouped_ffn(xs, row_gate, be, w1, b1, w2, b2, W_out, bo)
    return _sc_combine(ao, pos0, pos1, N, H)
